# baseline (speedup 1.0000x reference)
import jax
import jax.numpy as jnp
from jax.experimental import pallas as pl
from jax.experimental.pallas import tpu as pltpu

_NHEADS = 4
_NHID = 64
_ALPHA = 0.2
_NEG = -9e15
_LOG2E = 1.4426950408889634


def _elu(x):
    return jnp.where(x > 0, x, jnp.exp(x) - 1.0)


def _prep_kernel(raw_drone_ref, raw_dock_ref, W_pd_ref, b_pd_ref,
                 W_pk_ref, b_pk_ref, W_att_ref, A1_ref, A2_ref,
                 h_dock_ref, Whk_ref, skT_ref, sd_ref):
    h_dock = _elu(
        jnp.dot(raw_dock_ref[...], W_pk_ref[...],
                preferred_element_type=jnp.float32) + b_pk_ref[...])
    h_dock_ref[...] = h_dock
    sk = jnp.dot(h_dock, A2_ref[...], preferred_element_type=jnp.float32)
    skT_ref[...] = (sk.T * _LOG2E).astype(jnp.bfloat16)
    n_dock = h_dock.shape[0]
    for h in range(_NHEADS):
        whk = jnp.dot(h_dock, W_att_ref[h],
                      preferred_element_type=jnp.float32).astype(jnp.bfloat16)
        Whk_ref[h, :, 0:_NHID] = whk
        Whk_ref[h, :, _NHID:_NHID + 1] = jnp.ones((n_dock, 1), jnp.bfloat16)
        Whk_ref[h, :, _NHID + 1:] = jnp.zeros((n_dock, _NHID - 1), jnp.bfloat16)
    h_drone = _elu(
        jnp.dot(raw_drone_ref[...], W_pd_ref[...],
                preferred_element_type=jnp.float32) + b_pd_ref[...])
    sd_ref[...] = (jnp.dot(h_drone, A1_ref[...],
                           preferred_element_type=jnp.float32)
                   * _LOG2E).astype(jnp.bfloat16)


def _gat_block_kernel(adjT_ref, sd_ref, skT_ref, Whk_ref, W_fuse_ref,
                      b_fuse_ref, out_ref):
    maskT = jnp.where(adjT_ref[...] > 0, 0.0,
                      _NEG * _LOG2E).astype(jnp.bfloat16)
    maskf = maskT.T
    sd = sd_ref[...]
    acc = jnp.broadcast_to(b_fuse_ref[...], out_ref.shape)
    for h in range(_NHEADS):
        e = sd[:, h:h + 1] + skT_ref[h:h + 1, :]
        e = jnp.maximum(e, jnp.bfloat16(_ALPHA) * e)
        e = e + maskf
        m = jnp.max(e, axis=1, keepdims=True)
        p = jnp.exp2(e - m)
        aug = jnp.dot(p, Whk_ref[h],
                      preferred_element_type=jnp.float32)
        s = aug[:, _NHID:_NHID + 1]
        head = _elu(aug[:, 0:_NHID] * (1.0 / s))
        acc = acc + jnp.dot(head, W_fuse_ref[h],
                            preferred_element_type=jnp.float32)
    out_ref[...] = acc


@jax.jit
def kernel(raw_drone, raw_dock, adj, W_pd, b_pd, W_pk, b_pk, W_att, a_att,
           W_fuse, b_fuse):
    n_drone, nfeat_drone = raw_drone.shape
    n_dock, nfeat_dock = raw_dock.shape
    nheads, nhid, _ = W_att.shape

    A1 = jnp.einsum('hij,hj->ih', W_att, a_att[:, :nhid])
    A2 = jnp.einsum('hij,hj->ih', W_att, a_att[:, nhid:])
    W_fuse_h = W_fuse.reshape(nheads, nhid, nhid)
    b_pk2 = b_pk.reshape(1, nhid)
    b_pd2 = b_pd.reshape(1, nhid)
    b_fuse2 = b_fuse.reshape(1, nhid)

    h_dock, Whk, skT, sd = pl.pallas_call(
        _prep_kernel,
        out_shape=(
            jax.ShapeDtypeStruct((n_dock, nhid), jnp.float32),
            jax.ShapeDtypeStruct((nheads, n_dock, 2 * nhid), jnp.bfloat16),
            jax.ShapeDtypeStruct((nheads, n_dock), jnp.bfloat16),
            jax.ShapeDtypeStruct((n_drone, nheads), jnp.bfloat16),
        ),
    )(raw_drone, raw_dock, W_pd, b_pd2, W_pk, b_pk2, W_att, A1, A2)

    blk = 1536
    grid = (pl.cdiv(n_drone, blk),)
    out_drone = pl.pallas_call(
        _gat_block_kernel,
        grid=grid,
        in_specs=[
            pl.BlockSpec((n_dock, blk), lambda i: (0, i)),
            pl.BlockSpec((blk, nheads), lambda i: (i, 0)),
            pl.BlockSpec((nheads, n_dock), lambda i: (0, 0)),
            pl.BlockSpec((nheads, n_dock, 2 * nhid), lambda i: (0, 0, 0)),
            pl.BlockSpec((nheads, nhid, nhid), lambda i: (0, 0, 0)),
            pl.BlockSpec((1, nhid), lambda i: (0, 0)),
        ],
        out_specs=pl.BlockSpec((blk, nhid), lambda i: (i, 0)),
        out_shape=jax.ShapeDtypeStruct((n_drone, nhid), jnp.float32),
        compiler_params=pltpu.CompilerParams(
            dimension_semantics=("arbitrary",)),
    )(adj.T, sd, skT, Whk, W_fuse_h, b_fuse2)

    return (out_drone, h_dock)


# speedup vs baseline: 1.0827x; 1.0827x over previous
import jax
import jax.numpy as jnp
from jax.experimental import pallas as pl
from jax.experimental.pallas import tpu as pltpu

_NHEADS = 4
_NHID = 64
_ALPHA = 0.2
_NEG = -9e15
_LOG2E = 1.4426950408889634


def _elu(x):
    return jnp.where(x > 0, x, jnp.exp(x) - 1.0)


def _prep_kernel(raw_drone_ref, raw_dock_ref, W_pd_ref, b_pd_ref,
                 W_pk_ref, b_pk_ref, W_att_ref, A1_ref, A2_ref,
                 h_dock_ref, Whk_ref, skT_ref, sd_ref):
    h_dock = _elu(
        jnp.dot(raw_dock_ref[...], W_pk_ref[...],
                preferred_element_type=jnp.float32) + b_pk_ref[...])
    h_dock_ref[...] = h_dock
    sk = jnp.dot(h_dock, A2_ref[...], preferred_element_type=jnp.float32)
    skT_ref[...] = (sk.T * _LOG2E).astype(jnp.bfloat16)
    n_dock = h_dock.shape[0]
    for h in range(_NHEADS):
        whk = jnp.dot(h_dock, W_att_ref[h],
                      preferred_element_type=jnp.float32).astype(jnp.bfloat16)
        Whk_ref[h, :, 0:_NHID] = whk
        Whk_ref[h, :, _NHID:_NHID + 1] = jnp.ones((n_dock, 1), jnp.bfloat16)
        Whk_ref[h, :, _NHID + 1:] = jnp.zeros((n_dock, _NHID - 1), jnp.bfloat16)
    h_drone = _elu(
        jnp.dot(raw_drone_ref[...], W_pd_ref[...],
                preferred_element_type=jnp.float32) + b_pd_ref[...])
    sd_ref[...] = (jnp.dot(h_drone, A1_ref[...],
                           preferred_element_type=jnp.float32)
                   * _LOG2E).astype(jnp.bfloat16)


def _gat_block_kernel(adjT_ref, sd_ref, skT_ref, Whk_ref, W_fuse_ref,
                      b_fuse_ref, out_ref):
    maskT = jnp.where(adjT_ref[...] > 0, 0.0,
                      _NEG * _LOG2E).astype(jnp.bfloat16)
    maskf = maskT.T
    sd = sd_ref[...]
    acc = jnp.broadcast_to(b_fuse_ref[...], out_ref.shape)
    for h in range(_NHEADS):
        e = sd[:, h:h + 1] + skT_ref[h:h + 1, :]
        e = jnp.maximum(e, jnp.bfloat16(_ALPHA) * e)
        e = e + maskf
        m = jnp.max(e, axis=1, keepdims=True)
        p = jnp.exp2(e - m)
        aug = jnp.dot(p, Whk_ref[h],
                      preferred_element_type=jnp.float32)
        s = aug[:, _NHID:_NHID + 1]
        head = _elu(aug[:, 0:_NHID] * (1.0 / s)).astype(jnp.bfloat16)
        acc = acc + jnp.dot(head, W_fuse_ref[h],
                            preferred_element_type=jnp.float32)
    out_ref[...] = acc


@jax.jit
def kernel(raw_drone, raw_dock, adj, W_pd, b_pd, W_pk, b_pk, W_att, a_att,
           W_fuse, b_fuse):
    n_drone, nfeat_drone = raw_drone.shape
    n_dock, nfeat_dock = raw_dock.shape
    nheads, nhid, _ = W_att.shape

    A1 = jnp.einsum('hij,hj->ih', W_att, a_att[:, :nhid])
    A2 = jnp.einsum('hij,hj->ih', W_att, a_att[:, nhid:])
    W_fuse_h = W_fuse.reshape(nheads, nhid, nhid).astype(jnp.bfloat16)
    b_pk2 = b_pk.reshape(1, nhid)
    b_pd2 = b_pd.reshape(1, nhid)
    b_fuse2 = b_fuse.reshape(1, nhid)

    h_dock, Whk, skT, sd = pl.pallas_call(
        _prep_kernel,
        out_shape=(
            jax.ShapeDtypeStruct((n_dock, nhid), jnp.float32),
            jax.ShapeDtypeStruct((nheads, n_dock, 2 * nhid), jnp.bfloat16),
            jax.ShapeDtypeStruct((nheads, n_dock), jnp.bfloat16),
            jax.ShapeDtypeStruct((n_drone, nheads), jnp.bfloat16),
        ),
    )(raw_drone, raw_dock, W_pd, b_pd2, W_pk, b_pk2, W_att, A1, A2)

    blk = 1280
    grid = (pl.cdiv(n_drone, blk),)
    out_drone = pl.pallas_call(
        _gat_block_kernel,
        grid=grid,
        in_specs=[
            pl.BlockSpec((n_dock, blk), lambda i: (0, i)),
            pl.BlockSpec((blk, nheads), lambda i: (i, 0)),
            pl.BlockSpec((nheads, n_dock), lambda i: (0, 0)),
            pl.BlockSpec((nheads, n_dock, 2 * nhid), lambda i: (0, 0, 0)),
            pl.BlockSpec((nheads, nhid, nhid), lambda i: (0, 0, 0)),
            pl.BlockSpec((1, nhid), lambda i: (0, 0)),
        ],
        out_specs=pl.BlockSpec((blk, nhid), lambda i: (i, 0)),
        out_shape=jax.ShapeDtypeStruct((n_drone, nhid), jnp.float32),
        compiler_params=pltpu.CompilerParams(
            dimension_semantics=("arbitrary",)),
    )(adj.T, sd, skT, Whk, W_fuse_h, b_fuse2)

    return (out_drone, h_dock)


# speedup vs baseline: 1.0918x; 1.0085x over previous
import jax
import jax.numpy as jnp
from jax.experimental import pallas as pl
from jax.experimental.pallas import tpu as pltpu

_NHEADS = 4
_NHID = 64
_ALPHA = 0.2
_NEG = -9e15
_LOG2E = 1.4426950408889634


def _elu(x):
    return jnp.where(x > 0, x, jnp.exp(x) - 1.0)


def _prep_kernel(raw_drone_ref, raw_dock_ref, W_pd_ref, b_pd_ref,
                 W_pk_ref, b_pk_ref, W_att_ref, A1_ref, A2_ref,
                 h_dock_ref, Whk_ref, skT_ref, sd_ref):
    h_dock = _elu(
        jnp.dot(raw_dock_ref[...], W_pk_ref[...],
                preferred_element_type=jnp.float32) + b_pk_ref[...])
    h_dock_ref[...] = h_dock
    sk = jnp.dot(h_dock, A2_ref[...], preferred_element_type=jnp.float32)
    skT_ref[...] = (sk.T * _LOG2E).astype(jnp.bfloat16)
    n_dock = h_dock.shape[0]
    for h in range(_NHEADS):
        whk = jnp.dot(h_dock, W_att_ref[h],
                      preferred_element_type=jnp.float32).astype(jnp.bfloat16)
        Whk_ref[h, :, 0:_NHID] = whk
        Whk_ref[h, :, _NHID:_NHID + 1] = jnp.ones((n_dock, 1), jnp.bfloat16)
        Whk_ref[h, :, _NHID + 1:] = jnp.zeros((n_dock, _NHID - 1), jnp.bfloat16)
    h_drone = _elu(
        jnp.dot(raw_drone_ref[...], W_pd_ref[...],
                preferred_element_type=jnp.float32) + b_pd_ref[...])
    sd_ref[...] = (jnp.dot(h_drone, A1_ref[...],
                           preferred_element_type=jnp.float32)
                   * _LOG2E).astype(jnp.bfloat16)


def _gat_block_kernel(adjT_ref, sd_ref, skT_ref, Whk_ref, W_fuse_ref,
                      b_fuse_ref, out_ref):
    maskT = jnp.where(adjT_ref[...] > 0, 0.0,
                      _NEG * _LOG2E).astype(jnp.bfloat16)
    maskf = maskT.T
    sd = sd_ref[...]
    acc = jnp.broadcast_to(b_fuse_ref[...], out_ref.shape)
    for h in range(_NHEADS):
        e = sd[:, h:h + 1] + skT_ref[h:h + 1, :]
        e = jnp.maximum(e, jnp.bfloat16(_ALPHA) * e)
        e = e + maskf
        m = jnp.max(e, axis=1, keepdims=True)
        p = jnp.exp2(e - m)
        aug = jnp.dot(p, Whk_ref[h],
                      preferred_element_type=jnp.float32)
        s = aug[:, _NHID:_NHID + 1]
        head = _elu(aug[:, 0:_NHID] * (1.0 / s))
        acc = acc + jnp.dot(head, W_fuse_ref[h],
                            preferred_element_type=jnp.float32)
    out_ref[...] = acc


@jax.jit
def kernel(raw_drone, raw_dock, adj, W_pd, b_pd, W_pk, b_pk, W_att, a_att,
           W_fuse, b_fuse):
    n_drone, nfeat_drone = raw_drone.shape
    n_dock, nfeat_dock = raw_dock.shape
    nheads, nhid, _ = W_att.shape

    A1 = jnp.einsum('hij,hj->ih', W_att, a_att[:, :nhid])
    A2 = jnp.einsum('hij,hj->ih', W_att, a_att[:, nhid:])
    W_fuse_h = W_fuse.reshape(nheads, nhid, nhid)
    b_pk2 = b_pk.reshape(1, nhid)
    b_pd2 = b_pd.reshape(1, nhid)
    b_fuse2 = b_fuse.reshape(1, nhid)

    h_dock, Whk, skT, sd = pl.pallas_call(
        _prep_kernel,
        out_shape=(
            jax.ShapeDtypeStruct((n_dock, nhid), jnp.float32),
            jax.ShapeDtypeStruct((nheads, n_dock, 2 * nhid), jnp.bfloat16),
            jax.ShapeDtypeStruct((nheads, n_dock), jnp.bfloat16),
            jax.ShapeDtypeStruct((n_drone, nheads), jnp.bfloat16),
        ),
    )(raw_drone, raw_dock, W_pd, b_pd2, W_pk, b_pk2, W_att, A1, A2)

    blk = 1280
    grid = (pl.cdiv(n_drone, blk),)
    out_drone = pl.pallas_call(
        _gat_block_kernel,
        grid=grid,
        in_specs=[
            pl.BlockSpec((n_dock, blk), lambda i: (0, i)),
            pl.BlockSpec((blk, nheads), lambda i: (i, 0)),
            pl.BlockSpec((nheads, n_dock), lambda i: (0, 0)),
            pl.BlockSpec((nheads, n_dock, 2 * nhid), lambda i: (0, 0, 0)),
            pl.BlockSpec((nheads, nhid, nhid), lambda i: (0, 0, 0)),
            pl.BlockSpec((1, nhid), lambda i: (0, 0)),
        ],
        out_specs=pl.BlockSpec((blk, nhid), lambda i: (i, 0)),
        out_shape=jax.ShapeDtypeStruct((n_drone, nhid), jnp.float32),
        compiler_params=pltpu.CompilerParams(
            dimension_semantics=("arbitrary",)),
    )(adj.T, sd, skT, Whk, W_fuse_h, b_fuse2)

    return (out_drone, h_dock)
